# Initial kernel scaffold; baseline (speedup 1.0000x reference)
import jax
import jax.numpy as jnp
from jax.experimental import pallas as pl


def kernel(coords, feature, points):
    raise NotImplementedError("write your pallas kernel here")


# speedup vs baseline: 2.2745x; 2.2745x over previous
import functools

import jax
import jax.numpy as jnp
from jax import lax
from jax.experimental import pallas as pl
from jax.experimental.pallas import tpu as pltpu
from jax.experimental.pallas import tpu_sc as plsc

_LANES = 128
_BN = 2048

_SC_CORES = 2
_SC_SUBCORES = 16
_NW = _SC_CORES * _SC_SUBCORES


def _argmin_kernel_body(nblocks, points_ref, ct_ref, out_ref, bestd_ref,
                        besti_ref):
    j = pl.program_id(0)
    q = points_ref.shape[0]

    @pl.when(j == 0)
    def _init():
        bestd_ref[...] = jnp.full((q, _LANES), jnp.inf, jnp.float32)
        besti_ref[...] = jnp.zeros((q, _LANES), jnp.int32)

    px = points_ref[:, 0:1]
    py = points_ref[:, 1:2]
    pz = points_ref[:, 2:3]
    lane = lax.broadcasted_iota(jnp.int32, (q, _LANES), 1)
    for c in range(_BN // _LANES):
        cx = ct_ref[0:1, pl.ds(c * _LANES, _LANES)]
        cy = ct_ref[1:2, pl.ds(c * _LANES, _LANES)]
        cz = ct_ref[2:3, pl.ds(c * _LANES, _LANES)]
        dx = px - cx
        dy = py - cy
        dz = pz - cz
        d = dx * dx + dy * dy + dz * dz
        idx = lane + (j * _BN + c * _LANES)
        lt = d < bestd_ref[...]
        bestd_ref[...] = jnp.where(lt, d, bestd_ref[...])
        besti_ref[...] = jnp.where(lt, idx, besti_ref[...])

    @pl.when(j == nblocks - 1)
    def _final():
        bd = bestd_ref[...]
        bi = besti_ref[...]
        m = jnp.min(bd, axis=1, keepdims=True)
        cand = jnp.where(bd == m, bi, jnp.int32(2**31 - 1))
        out_ref[...] = jnp.min(cand, axis=1, keepdims=True)


@functools.lru_cache(maxsize=None)
def _make_argmin(q, npad):
    nblocks = npad // _BN
    return pl.pallas_call(
        functools.partial(_argmin_kernel_body, nblocks),
        grid=(nblocks,),
        in_specs=[
            pl.BlockSpec((q, 3), lambda j: (0, 0)),
            pl.BlockSpec((3, _BN), lambda j: (0, j)),
        ],
        out_specs=pl.BlockSpec((q, 1), lambda j: (0, 0)),
        out_shape=jax.ShapeDtypeStruct((q, 1), jnp.int32),
        scratch_shapes=[
            pltpu.VMEM((q, _LANES), jnp.float32),
            pltpu.VMEM((q, _LANES), jnp.int32),
        ],
        compiler_params=pltpu.CompilerParams(
            dimension_semantics=("arbitrary",)),
    )


@functools.lru_cache(maxsize=None)
def _make_sc_gather(n, d, q):
    bpw = q // _NW
    mesh = plsc.VectorSubcoreMesh(core_axis_name="c", subcore_axis_name="s")

    @functools.partial(
        pl.kernel,
        mesh=mesh,
        out_type=jax.ShapeDtypeStruct((q, d), jnp.float32),
        scratch_types=[
            pltpu.VMEM((bpw,), jnp.int32),
            pltpu.VMEM((bpw, d), jnp.float32),
            pltpu.SemaphoreType.DMA,
        ],
        compiler_params=pltpu.CompilerParams(use_tc_tiling_on_sc=False),
    )
    def _gather(table_hbm, idx_hbm, out_hbm, idx_v, rows_v, sem):
        wid = lax.axis_index("s") * _SC_CORES + lax.axis_index("c")
        base = wid * bpw
        pltpu.sync_copy(idx_hbm.at[pl.ds(base, bpw)], idx_v)
        pltpu.async_copy(table_hbm.at[idx_v], rows_v, sem).wait()
        pltpu.sync_copy(rows_v, out_hbm.at[pl.ds(base, bpw)])

    return _gather


def kernel(coords, feature, points):
    n, _ = coords.shape
    q, _ = points.shape
    d = feature.shape[1]
    npad = ((n + _BN - 1) // _BN) * _BN
    ct = jnp.pad(coords.T, ((0, 0), (0, npad - n)),
                 constant_values=jnp.inf)
    idx = _make_argmin(q, npad)(points, ct).reshape(q)
    return _make_sc_gather(n, d, q)(feature, idx)
